# baseline (speedup 1.0000x reference)
import functools

import jax
import jax.numpy as jnp
from jax import lax
from jax.experimental import pallas as pl
from jax.experimental.pallas import tpu as pltpu
from jax.experimental.pallas import tpu_sc as plsc

ATTR_W = 1.0
REPL_W = 1.0
MARGIN_W = 5.0
THR = 0.5
MARG = 0.2

NUM_INST = 64
ROW_TILE = 256
LOG2E = 1.4426950408889634


def _make_compact_sc(b, n, d):
    mesh = plsc.VectorSubcoreMesh(core_axis_name="c", subcore_axis_name="s")

    @functools.partial(
        pl.kernel,
        mesh=mesh,
        out_type=(
            jax.ShapeDtypeStruct((b * n, d), jnp.float32),
            jax.ShapeDtypeStruct((b, 16), jnp.int32),
        ),
        scratch_types=[
            pltpu.VMEM((n,), jnp.int32),
            pltpu.VMEM((n,), jnp.int32),
            pltpu.VMEM((n, d), jnp.float32),
            pltpu.VMEM((16,), jnp.int32),
            pltpu.SemaphoreType.DMA,
        ],
        compiler_params=pltpu.CompilerParams(use_tc_tiling_on_sc=False),
    )
    def compact(iscp_hbm, emb_hbm, rows_out, cnt_out, flags_v, dst_v, rows_v,
                cnt_v, sem):
        num_cores = 2
        wid = lax.axis_index("s") * num_cores + lax.axis_index("c")

        @pl.when(wid < b)
        def _():
            base = wid * n
            pltpu.sync_copy(iscp_hbm.at[pl.ds(base, n)], flags_v)
            lane = lax.iota(jnp.int32, 16)

            def shift_add(v, k):
                sh = v.at[jnp.maximum(lane - k, 0)].get(
                    mode="promise_in_bounds")
                return v + jnp.where(lane >= k, sh, 0.0)

            def body(i, carry_v):
                f = flags_v[pl.ds(i * 16, 16)].astype(jnp.float32)
                csum = f
                for k in (1, 2, 4, 8):
                    csum = shift_add(csum, k)
                pos = ((carry_v + csum) - f).astype(jnp.int32)
                dst_v[pl.ds(i * 16, 16)] = jnp.where(
                    f > 0.0, base + pos, base + (n - 1))
                tot = csum.at[jnp.full((16,), 15, jnp.int32)].get(
                    mode="promise_in_bounds")
                return carry_v + tot

            cnt_f = lax.fori_loop(0, n // 16, body,
                                  jnp.zeros((16,), jnp.float32))
            cnt_v[...] = cnt_f.astype(jnp.int32)
            pltpu.sync_copy(cnt_v, cnt_out.at[wid])
            pltpu.sync_copy(emb_hbm.at[pl.ds(base, n)], rows_v)
            pltpu.async_copy(rows_v, rows_out.at[dst_v], sem).wait()

    return compact


def _oc_kernel(cnt_ref, beta_ref, emb_ref, sid_ref, iscp_ref, iscp_col_ref,
               comp_ref, out_ref, acc_sm):
    n = emb_ref.shape[1]
    d = emb_ref.shape[2]
    nb = pl.num_programs(0)
    x = beta_ref[0]
    emb = emb_ref[0]
    sid = sid_ref[0]
    iscp = iscp_ref[0]
    pos_col = (iscp_col_ref[0] == 1).astype(jnp.float32)
    comp = comp_ref[0]
    i = pl.program_id(0)
    p_cnt = cnt_ref[i]

    pos = (iscp == 1).astype(jnp.float32)
    neg = 1.0 - pos
    pos_count = jnp.sum(pos)
    neg_count = jnp.sum(neg)
    valid = jnp.where((pos_count >= 1.0) & (neg_count >= 1.0), 1.0, 0.0)

    pw = neg_count / (pos_count + 1e-6)
    sp_neg = jax.nn.softplus(-x)
    sp_pos = jax.nn.softplus(x)
    bce = jnp.sum(pw * pos * sp_neg + neg * sp_pos) / n
    prob = jax.nn.sigmoid(x)
    pos_m = jnp.sum(jax.nn.relu(THR + MARG - prob) * pos) / jnp.maximum(pos_count, 1.0)
    neg_m = jnp.sum(jax.nn.relu(prob - (THR - MARG)) * neg) / jnp.maximum(neg_count, 1.0)
    beta_loss = bce + MARGIN_W * (pos_m + neg_m)

    n2c = jnp.sum(emb * emb, axis=1, keepdims=True)
    inst_iota = jax.lax.broadcasted_iota(jnp.int32, (NUM_INST, n), 0)
    m = (sid == inst_iota).astype(jnp.float32)
    ones_col = jnp.ones((n, 1), jnp.float32)
    emb_aug = jnp.concatenate([emb, n2c, ones_col], axis=1)
    s_aug = jnp.dot(m, emb_aug, preferred_element_type=jnp.float32)
    s1 = s_aug[:, :d]
    s2 = s_aug[:, d]
    cnt = s_aug[:, d + 1]

    col_iota = jax.lax.broadcasted_iota(jnp.int32, (NUM_INST, n), 1)
    cp_inst = (m > 0.0) & (pos > 0.0)
    first = jnp.min(jnp.where(cp_inst, col_iota, n), axis=1)
    has_cp = first < n
    first_c = jnp.where(has_cp, first, 0)
    pick = (col_iota == first_c[:, None]).astype(jnp.float32)
    cp_ref = jnp.dot(pick, emb, preferred_element_type=jnp.float32)
    cp_n2 = jnp.sum(cp_ref * cp_ref, axis=1)

    mean_d2 = (s2 - 2.0 * jnp.sum(cp_ref * s1, axis=1) + cnt * cp_n2) / jnp.maximum(cnt, 1.0)
    attraction = jnp.sum(jnp.where(has_cp, mean_d2, 0.0)) * ATTR_W

    row_iota_col = jax.lax.broadcasted_iota(jnp.int32, (n, 1), 0)
    live = row_iota_col < p_cnt
    comp = jnp.where(live, comp, 0.0)
    c2c = jnp.sum(comp * comp, axis=1, keepdims=True)
    c_shift = 0.5 * jnp.max(c2c)
    cs = comp * jnp.float32((2.0 * LOG2E) ** 0.5)
    nc = jnp.where(live, LOG2E * (c2c + c_shift), jnp.float32(1.0e9))
    lu = jnp.where(live, LOG2E * (c_shift - c2c), jnp.float32(-1.0e9))
    row_aug = jnp.concatenate([cs, lu, ones_col], axis=1)
    col_aug = jnp.concatenate([cs, ones_col, -nc], axis=1)

    ri = jax.lax.broadcasted_iota(jnp.int32, (ROW_TILE, ROW_TILE), 0)
    ci = jax.lax.broadcasted_iota(jnp.int32, (ROW_TILE, ROW_TILE), 1)
    diag_mask = (ci > ri).astype(jnp.float32)

    acc_sm[0] = 0.0
    for t in range(n // ROW_TILE):
        for u in range(t, n // ROW_TILE):
            @pl.when(u * ROW_TILE < p_cnt)
            def _(t=t, u=u):
                et = row_aug[t * ROW_TILE:(t + 1) * ROW_TILE, :]
                ec = col_aug[u * ROW_TILE:(u + 1) * ROW_TILE, :]
                ex = jax.lax.dot_general(et, ec, (((1,), (1,)), ((), ())),
                                         preferred_element_type=jnp.float32)
                w = jnp.exp2(ex)
                if u == t:
                    w = w * diag_mask
                acc_sm[0] = acc_sm[0] + jnp.sum(w)

    pair_sum = pos_count + 2.0 * acc_sm[0]
    repulsion = jnp.where(
        pos_count > 1.0,
        pair_sum / jnp.maximum(pos_count * pos_count, 1.0),
        0.0,
    ) * REPL_W

    lane = jax.lax.broadcasted_iota(jnp.int32, (1, 128), 1)
    total_b = valid * (beta_loss + attraction + repulsion)
    outvec = (
        jnp.where(lane == 0, total_b, 0.0)
        + jnp.where(lane == 1, valid * beta_loss, 0.0)
        + jnp.where(lane == 2, valid * attraction, 0.0)
        + jnp.where(lane == 3, valid * repulsion, 0.0)
        + jnp.where(lane == 4, valid, 0.0)
    )

    @pl.when(i == 0)
    def _():
        out_ref[0] = outvec

    @pl.when(i > 0)
    def _():
        acc = out_ref[0] + outvec

        @pl.when(i == nb - 1)
        def _():
            count = jnp.sum(jnp.where(lane == 4, acc, 0.0))
            denom = jnp.maximum(count, 1.0)
            scale = jnp.where(count > 0.0, 1.0 / denom, 0.0)
            out_ref[0] = acc * scale

        @pl.when(i < nb - 1)
        def _():
            out_ref[0] = acc


@functools.partial(jax.jit, static_argnames=())
def kernel(beta, embed, slice_id, is_cp):
    b, n, d = embed.shape
    beta_s = jnp.reshape(beta, (b, 1, n))
    sid = jnp.reshape(slice_id.astype(jnp.int32), (b, 1, n))
    iscp = jnp.reshape(is_cp.astype(jnp.int32), (b, 1, n))
    iscp_col = jnp.reshape(is_cp.astype(jnp.int32), (b, n, 1))

    iscp_flat = jnp.reshape(is_cp.astype(jnp.int32), (b * n,))
    emb_flat = jnp.reshape(embed, (b * n, d))
    comp_flat, cnt16 = _make_compact_sc(b, n, d)(iscp_flat, emb_flat)
    comp = jnp.reshape(comp_flat, (b, n, d))
    counts = cnt16[:, 0]

    acc = pl.pallas_call(
        _oc_kernel,
        grid_spec=pltpu.PrefetchScalarGridSpec(
            num_scalar_prefetch=1,
            grid=(b,),
            in_specs=[
                pl.BlockSpec((1, 1, n), lambda i, c: (i, 0, 0)),
                pl.BlockSpec((1, n, d), lambda i, c: (i, 0, 0)),
                pl.BlockSpec((1, 1, n), lambda i, c: (i, 0, 0)),
                pl.BlockSpec((1, 1, n), lambda i, c: (i, 0, 0)),
                pl.BlockSpec((1, n, 1), lambda i, c: (i, 0, 0)),
                pl.BlockSpec((1, n, d), lambda i, c: (i, 0, 0)),
            ],
            out_specs=pl.BlockSpec((1, 1, 128), lambda i, c: (0, 0, 0)),
            scratch_shapes=[pltpu.SMEM((1,), jnp.float32)],
        ),
        out_shape=jax.ShapeDtypeStruct((1, 1, 128), jnp.float32),
    )(counts, beta_s, embed, sid, iscp, iscp_col, comp)

    v = acc[0, 0]
    return (v[0], v[1], v[2], v[3])


# speedup vs baseline: 2.4783x; 2.4783x over previous
import functools

import jax
import jax.numpy as jnp
from jax.experimental import pallas as pl

ATTR_W = 1.0
REPL_W = 1.0
MARGIN_W = 5.0
THR = 0.5
MARG = 0.2

NUM_INST = 64
ROW_TILE = 512
LOG2E = 1.4426950408889634


def _oc_kernel(beta_ref, emb_ref, sid_ref, iscp_ref, iscp_col_ref, out_ref):
    n = emb_ref.shape[1]
    d = emb_ref.shape[2]
    nb = pl.num_programs(0)
    x = beta_ref[0]
    emb = emb_ref[0]
    sid = sid_ref[0]
    iscp = iscp_ref[0]
    pos_col = (iscp_col_ref[0] == 1).astype(jnp.float32)

    pos = (iscp == 1).astype(jnp.float32)
    neg = 1.0 - pos
    pos_count = jnp.sum(pos)
    neg_count = jnp.sum(neg)
    valid = jnp.where((pos_count >= 1.0) & (neg_count >= 1.0), 1.0, 0.0)

    pw = neg_count / (pos_count + 1e-6)
    sp_neg = jax.nn.softplus(-x)
    sp_pos = jax.nn.softplus(x)
    bce = jnp.sum(pw * pos * sp_neg + neg * sp_pos) / n
    prob = jax.nn.sigmoid(x)
    pos_m = jnp.sum(jax.nn.relu(THR + MARG - prob) * pos) / jnp.maximum(pos_count, 1.0)
    neg_m = jnp.sum(jax.nn.relu(prob - (THR - MARG)) * neg) / jnp.maximum(neg_count, 1.0)
    beta_loss = bce + MARGIN_W * (pos_m + neg_m)

    n2c = jnp.sum(emb * emb, axis=1, keepdims=True)
    inst_iota = jax.lax.broadcasted_iota(jnp.int32, (NUM_INST, n), 0)
    m = (sid == inst_iota).astype(jnp.float32)
    ones_col = jnp.ones((n, 1), jnp.float32)
    emb_aug = jnp.concatenate([emb, n2c, ones_col], axis=1)
    s_aug = jnp.dot(m, emb_aug, preferred_element_type=jnp.float32)
    s1 = s_aug[:, :d]
    s2 = s_aug[:, d]
    cnt = s_aug[:, d + 1]

    col_iota = jax.lax.broadcasted_iota(jnp.int32, (NUM_INST, n), 1)
    cp_inst = (m > 0.0) & (pos > 0.0)
    first = jnp.min(jnp.where(cp_inst, col_iota, n), axis=1)
    has_cp = first < n
    first_c = jnp.where(has_cp, first, 0)
    pick = (col_iota == first_c[:, None]).astype(jnp.float32)
    cp_ref = jnp.dot(pick, emb, preferred_element_type=jnp.float32)
    cp_n2 = jnp.sum(cp_ref * cp_ref, axis=1)

    mean_d2 = (s2 - 2.0 * jnp.sum(cp_ref * s1, axis=1) + cnt * cp_n2) / jnp.maximum(cnt, 1.0)
    attraction = jnp.sum(jnp.where(has_cp, mean_d2, 0.0)) * ATTR_W

    c_shift = 0.5 * jnp.max(n2c)
    es = emb * jnp.float32((2.0 * LOG2E) ** 0.5)
    nc = jnp.where(pos_col > 0.0, LOG2E * (n2c + c_shift), jnp.float32(1.0e9))
    lu = jnp.where(pos_col > 0.0, LOG2E * (c_shift - n2c), jnp.float32(-1.0e9))
    row_aug = jnp.concatenate([es, lu, ones_col], axis=1)
    col_aug = jnp.concatenate([es, ones_col, -nc], axis=1)

    ri = jax.lax.broadcasted_iota(jnp.int32, (ROW_TILE, ROW_TILE), 0)
    ci = jax.lax.broadcasted_iota(jnp.int32, (ROW_TILE, ROW_TILE), 1)
    diag_mask = (ci > ri).astype(jnp.float32)

    upper = jnp.float32(0.0)
    for t in range(n // ROW_TILE):
        c0 = t * ROW_TILE
        et = row_aug[c0:c0 + ROW_TILE, :]
        ec = col_aug[c0:, :]
        ex = jax.lax.dot_general(et, ec, (((1,), (1,)), ((), ())),
                                 preferred_element_type=jnp.float32)
        w = jnp.exp2(ex)
        upper = upper + jnp.sum(w[:, :ROW_TILE] * diag_mask)
        if c0 + ROW_TILE < n:
            upper = upper + jnp.sum(w[:, ROW_TILE:])

    pair_sum = pos_count + 2.0 * upper
    repulsion = jnp.where(
        pos_count > 1.0,
        pair_sum / jnp.maximum(pos_count * pos_count, 1.0),
        0.0,
    ) * REPL_W

    lane = jax.lax.broadcasted_iota(jnp.int32, (1, 128), 1)
    total_b = valid * (beta_loss + attraction + repulsion)
    outvec = (
        jnp.where(lane == 0, total_b, 0.0)
        + jnp.where(lane == 1, valid * beta_loss, 0.0)
        + jnp.where(lane == 2, valid * attraction, 0.0)
        + jnp.where(lane == 3, valid * repulsion, 0.0)
        + jnp.where(lane == 4, valid, 0.0)
    )

    i = pl.program_id(0)

    @pl.when(i == 0)
    def _():
        out_ref[0] = outvec

    @pl.when(i > 0)
    def _():
        acc = out_ref[0] + outvec

        @pl.when(i == nb - 1)
        def _():
            count = jnp.sum(jnp.where(lane == 4, acc, 0.0))
            denom = jnp.maximum(count, 1.0)
            scale = jnp.where(count > 0.0, 1.0 / denom, 0.0)
            out_ref[0] = acc * scale

        @pl.when(i < nb - 1)
        def _():
            out_ref[0] = acc


@functools.partial(jax.jit, static_argnames=())
def kernel(beta, embed, slice_id, is_cp):
    b, n, d = embed.shape
    beta_s = jnp.reshape(beta, (b, 1, n))
    sid = jnp.reshape(slice_id.astype(jnp.int32), (b, 1, n))
    iscp = jnp.reshape(is_cp.astype(jnp.int32), (b, 1, n))
    iscp_col = jnp.reshape(is_cp.astype(jnp.int32), (b, n, 1))

    acc = pl.pallas_call(
        _oc_kernel,
        grid=(b,),
        in_specs=[
            pl.BlockSpec((1, 1, n), lambda i: (i, 0, 0)),
            pl.BlockSpec((1, n, d), lambda i: (i, 0, 0)),
            pl.BlockSpec((1, 1, n), lambda i: (i, 0, 0)),
            pl.BlockSpec((1, 1, n), lambda i: (i, 0, 0)),
            pl.BlockSpec((1, n, 1), lambda i: (i, 0, 0)),
        ],
        out_specs=pl.BlockSpec((1, 1, 128), lambda i: (0, 0, 0)),
        out_shape=jax.ShapeDtypeStruct((1, 1, 128), jnp.float32),
    )(beta_s, embed, sid, iscp, iscp_col)

    v = acc[0, 0]
    return (v[0], v[1], v[2], v[3])


# speedup vs baseline: 2.6163x; 1.0557x over previous
import functools

import jax
import jax.numpy as jnp
from jax.experimental import pallas as pl

ATTR_W = 1.0
REPL_W = 1.0
MARGIN_W = 5.0
THR = 0.5
MARG = 0.2

NUM_INST = 64
ROW_TILE = 128
LOG2E = 1.4426950408889634


def _oc_kernel(beta_ref, emb_ref, sid_ref, iscp_ref, iscp_col_ref, out_ref):
    n = emb_ref.shape[1]
    d = emb_ref.shape[2]
    nb = pl.num_programs(0)
    x = beta_ref[0]
    emb = emb_ref[0]
    sid = sid_ref[0]
    iscp = iscp_ref[0]
    pos_col = (iscp_col_ref[0] == 1).astype(jnp.float32)

    pos = (iscp == 1).astype(jnp.float32)
    neg = 1.0 - pos
    pos_count = jnp.sum(pos)
    neg_count = jnp.sum(neg)
    valid = jnp.where((pos_count >= 1.0) & (neg_count >= 1.0), 1.0, 0.0)

    pw = neg_count / (pos_count + 1e-6)
    sp_neg = jax.nn.softplus(-x)
    sp_pos = jax.nn.softplus(x)
    bce = jnp.sum(pw * pos * sp_neg + neg * sp_pos) / n
    prob = jax.nn.sigmoid(x)
    pos_m = jnp.sum(jax.nn.relu(THR + MARG - prob) * pos) / jnp.maximum(pos_count, 1.0)
    neg_m = jnp.sum(jax.nn.relu(prob - (THR - MARG)) * neg) / jnp.maximum(neg_count, 1.0)
    beta_loss = bce + MARGIN_W * (pos_m + neg_m)

    n2c = jnp.sum(emb * emb, axis=1, keepdims=True)
    inst_iota = jax.lax.broadcasted_iota(jnp.int32, (NUM_INST, n), 0)
    m = (sid == inst_iota).astype(jnp.float32)
    ones_col = jnp.ones((n, 1), jnp.float32)
    emb_aug = jnp.concatenate([emb, n2c, ones_col], axis=1)
    s_aug = jnp.dot(m, emb_aug, preferred_element_type=jnp.float32)
    s1 = s_aug[:, :d]
    s2 = s_aug[:, d]
    cnt = s_aug[:, d + 1]

    col_iota = jax.lax.broadcasted_iota(jnp.int32, (NUM_INST, n), 1)
    cp_inst = (m > 0.0) & (pos > 0.0)
    first = jnp.min(jnp.where(cp_inst, col_iota, n), axis=1)
    has_cp = first < n
    first_c = jnp.where(has_cp, first, 0)
    pick = (col_iota == first_c[:, None]).astype(jnp.float32)
    cp_ref = jnp.dot(pick, emb, preferred_element_type=jnp.float32)
    cp_n2 = jnp.sum(cp_ref * cp_ref, axis=1)

    mean_d2 = (s2 - 2.0 * jnp.sum(cp_ref * s1, axis=1) + cnt * cp_n2) / jnp.maximum(cnt, 1.0)
    attraction = jnp.sum(jnp.where(has_cp, mean_d2, 0.0)) * ATTR_W

    c_shift = 0.5 * jnp.max(n2c)
    es = emb * jnp.float32((2.0 * LOG2E) ** 0.5)
    nc = jnp.where(pos_col > 0.0, LOG2E * (n2c + c_shift), jnp.float32(1.0e9))
    lu = jnp.where(pos_col > 0.0, LOG2E * (c_shift - n2c), jnp.float32(-1.0e9))
    row_aug = jnp.concatenate([es, lu, ones_col], axis=1)
    col_aug = jnp.concatenate([es, ones_col, -nc], axis=1)

    ri = jax.lax.broadcasted_iota(jnp.int32, (ROW_TILE, ROW_TILE), 0)
    ci = jax.lax.broadcasted_iota(jnp.int32, (ROW_TILE, ROW_TILE), 1)
    diag_mask = (ci > ri).astype(jnp.float32)

    upper = jnp.float32(0.0)
    for t in range(n // ROW_TILE):
        c0 = t * ROW_TILE
        et = row_aug[c0:c0 + ROW_TILE, :]
        ec = col_aug[c0:, :]
        ex = jax.lax.dot_general(et, ec, (((1,), (1,)), ((), ())),
                                 preferred_element_type=jnp.float32)
        w = jnp.exp2(ex)
        upper = upper + jnp.sum(w[:, :ROW_TILE] * diag_mask)
        if c0 + ROW_TILE < n:
            upper = upper + jnp.sum(w[:, ROW_TILE:])

    pair_sum = pos_count + 2.0 * upper
    repulsion = jnp.where(
        pos_count > 1.0,
        pair_sum / jnp.maximum(pos_count * pos_count, 1.0),
        0.0,
    ) * REPL_W

    lane = jax.lax.broadcasted_iota(jnp.int32, (1, 128), 1)
    total_b = valid * (beta_loss + attraction + repulsion)
    outvec = (
        jnp.where(lane == 0, total_b, 0.0)
        + jnp.where(lane == 1, valid * beta_loss, 0.0)
        + jnp.where(lane == 2, valid * attraction, 0.0)
        + jnp.where(lane == 3, valid * repulsion, 0.0)
        + jnp.where(lane == 4, valid, 0.0)
    )

    i = pl.program_id(0)

    @pl.when(i == 0)
    def _():
        out_ref[0] = outvec

    @pl.when(i > 0)
    def _():
        acc = out_ref[0] + outvec

        @pl.when(i == nb - 1)
        def _():
            count = jnp.sum(jnp.where(lane == 4, acc, 0.0))
            denom = jnp.maximum(count, 1.0)
            scale = jnp.where(count > 0.0, 1.0 / denom, 0.0)
            out_ref[0] = acc * scale

        @pl.when(i < nb - 1)
        def _():
            out_ref[0] = acc


@functools.partial(jax.jit, static_argnames=())
def kernel(beta, embed, slice_id, is_cp):
    b, n, d = embed.shape
    beta_s = jnp.reshape(beta, (b, 1, n))
    sid = jnp.reshape(slice_id.astype(jnp.int32), (b, 1, n))
    iscp = jnp.reshape(is_cp.astype(jnp.int32), (b, 1, n))
    iscp_col = jnp.reshape(is_cp.astype(jnp.int32), (b, n, 1))

    acc = pl.pallas_call(
        _oc_kernel,
        grid=(b,),
        in_specs=[
            pl.BlockSpec((1, 1, n), lambda i: (i, 0, 0)),
            pl.BlockSpec((1, n, d), lambda i: (i, 0, 0)),
            pl.BlockSpec((1, 1, n), lambda i: (i, 0, 0)),
            pl.BlockSpec((1, 1, n), lambda i: (i, 0, 0)),
            pl.BlockSpec((1, n, 1), lambda i: (i, 0, 0)),
        ],
        out_specs=pl.BlockSpec((1, 1, 128), lambda i: (0, 0, 0)),
        out_shape=jax.ShapeDtypeStruct((1, 1, 128), jnp.float32),
    )(beta_s, embed, sid, iscp, iscp_col)

    v = acc[0, 0]
    return (v[0], v[1], v[2], v[3])
